# baseline (speedup 1.0000x reference)
import functools

import numpy as np
import jax
import jax.numpy as jnp
from jax import lax
from jax.experimental import pallas as pl
from jax.experimental.pallas import tpu as pltpu
from jax.experimental.pallas import tpu_sc as plsc

_PRIOR_BOXES = np.array([[1.3221, 1.73145], [3.19275, 4.00944], [5.05587, 8.09892],
                         [9.47112, 4.84053], [11.2364, 10.0071]], dtype=np.float32) / 13.0
_IMG_W = 416.0
_IMG_H = 416.0
_LAMBDA_OBJ = 5.0
_LAMBDA_NONOBJ = 0.5

_B, _C, _H, _W = 64, 125, 52, 52
_HW = _H * _W
_NV = _HW // 16
_K = 5
_CHW = _K * _HW
_NCH = _C // _K
_NWORK = 32
_BPW = _B // _NWORK


def _grid_vecs():
    dx = np.float32(_IMG_W / _C)
    dy = np.float32(_IMG_H / _C)
    cell_x = np.tile(np.arange(_W, dtype=np.float32), _H)
    cell_y = np.repeat(np.arange(_H, dtype=np.float32), _W)
    return np.concatenate([dx * cell_x, dy * cell_y]), dx, dy


def _sc_body(dx, dy, pred_hbm, y0_hbm, gxy_hbm, out_hbm,
             gx_v, gy_v, mask_v, in0, in1, out0, out1,
             sin0, sin1, sout0, sout1):
    wid = lax.axis_index("s") * 2 + lax.axis_index("c")
    pltpu.sync_copy(gxy_hbm.at[pl.ds(0, _HW)], gx_v)
    pltpu.sync_copy(gxy_hbm.at[pl.ds(_HW, _HW)], gy_v)
    in_bufs, in_sems = (in0, in1), (sin0, sin1)
    out_bufs, out_sems = (out0, out1), (sout0, sout1)

    for bi in range(_BPW):
        b = wid * _BPW + bi
        bbase = pl.multiple_of(b * (_C * _HW), 8)
        ybase = pl.multiple_of(b * _HW, 8)

        pltpu.sync_copy(y0_hbm.at[pl.ds(ybase, _HW)], mask_v)

        def mask_body(v, _):
            sl = pl.ds(v * 16, 16)
            y = mask_v[sl]
            mask_v[sl] = _LAMBDA_OBJ * y + _LAMBDA_NONOBJ * jnp.negative(y + (-1.0))
            return 0

        lax.fori_loop(0, _NV, mask_body, 0)

        def in_cp(ch, buf, sem):
            return pltpu.make_async_copy(
                pred_hbm.at[pl.ds(bbase + ch * _CHW, _CHW)], buf, sem)

        def out_cp(ch, buf, sem):
            return pltpu.make_async_copy(
                buf, out_hbm.at[pl.ds(bbase + ch * _CHW, _CHW)], sem)

        in_cp(0, in_bufs[0], in_sems[0]).start()
        for ch in range(_NCH):
            cur = ch % 2
            in_cp(ch, in_bufs[cur], in_sems[cur]).wait()
            if ch + 1 < _NCH:
                in_cp(ch + 1, in_bufs[1 - cur], in_sems[1 - cur]).start()
            if ch >= 2:
                out_cp(ch - 2, out_bufs[cur], out_sems[cur]).wait()
            ib, ob = in_bufs[cur], out_bufs[cur]
            if ch % 5 == 0:
                pw = float(_PRIOR_BOXES[ch // 5, 0])
                ph = float(_PRIOR_BOXES[ch // 5, 1])

                def body(v, _, ib=ib, ob=ob, pw=pw, ph=ph):
                    o = v * 16
                    s0 = pl.ds(o, 16)
                    s1 = pl.ds(_HW + o, 16)
                    s2 = pl.ds(2 * _HW + o, 16)
                    s3 = pl.ds(3 * _HW + o, 16)
                    s4 = pl.ds(4 * _HW + o, 16)
                    m = mask_v[s0]
                    gxv = gx_v[s0]
                    gyv = gy_v[s0]
                    ob[s0] = ib[s0] * m
                    t1 = (dx * ib[s1]).astype(jnp.int32).astype(jnp.float32)
                    ob[s1] = (t1 + gxv) * m
                    t2 = (dy * ib[s2]).astype(jnp.int32).astype(jnp.float32)
                    ob[s2] = (t2 + gyv) * m
                    t3 = ((pw * ib[s3]) * _IMG_W).astype(jnp.int32).astype(jnp.float32)
                    ob[s3] = t3 * m
                    t4 = ((ph * ib[s4]) * _IMG_H).astype(jnp.int32).astype(jnp.float32)
                    ob[s4] = t4 * m
                    return 0
            else:
                def body(v, _, ib=ib, ob=ob):
                    o = v * 16
                    m = mask_v[pl.ds(o, 16)]
                    for j in range(_K):
                        sj = pl.ds(j * _HW + o, 16)
                        ob[sj] = ib[sj] * m
                    return 0

            lax.fori_loop(0, _NV, body, 0)
            out_cp(ch, out_bufs[cur], out_sems[cur]).start()
        out_cp(_NCH - 2, out_bufs[1], out_sems[1]).wait()
        out_cp(_NCH - 1, out_bufs[0], out_sems[0]).wait()


def kernel(pred, y_hat):
    B, C, H, W = pred.shape
    HW = H * W
    gxy, dx, dy = _grid_vecs()

    pred1 = pred.reshape(B * C * HW)
    y0 = y_hat[:, :, :, 0].reshape(B * HW)

    mesh = plsc.VectorSubcoreMesh(core_axis_name="c", subcore_axis_name="s")
    sc = pl.kernel(
        functools.partial(_sc_body, dx, dy),
        mesh=mesh,
        out_type=jax.ShapeDtypeStruct((B * C * HW,), jnp.float32),
        scratch_types=[
            pltpu.VMEM((_HW,), jnp.float32),
            pltpu.VMEM((_HW,), jnp.float32),
            pltpu.VMEM((_HW,), jnp.float32),
            pltpu.VMEM((_CHW,), jnp.float32),
            pltpu.VMEM((_CHW,), jnp.float32),
            pltpu.VMEM((_CHW,), jnp.float32),
            pltpu.VMEM((_CHW,), jnp.float32),
            pltpu.SemaphoreType.DMA,
            pltpu.SemaphoreType.DMA,
            pltpu.SemaphoreType.DMA,
            pltpu.SemaphoreType.DMA,
        ],
    )
    out = sc(pred1, y0, jnp.asarray(gxy))
    return out.reshape(B, C, H, W)


# speedup vs baseline: 2.4735x; 2.4735x over previous
import functools

import numpy as np
import jax
import jax.numpy as jnp
from jax import lax
from jax.experimental import pallas as pl
from jax.experimental.pallas import tpu as pltpu
from jax.experimental.pallas import tpu_sc as plsc

_PRIOR_BOXES = np.array([[1.3221, 1.73145], [3.19275, 4.00944], [5.05587, 8.09892],
                         [9.47112, 4.84053], [11.2364, 10.0071]], dtype=np.float32) / 13.0
_IMG_W = 416.0
_IMG_H = 416.0
_LAMBDA_OBJ = 5.0
_LAMBDA_NONOBJ = 0.5

_B, _C, _H, _W = 64, 125, 52, 52
_HW = _H * _W
_NV = _HW // 16
_K = 8
_NWORK = 32
_BPW = _B // _NWORK
_CHUNKS = [(c0, min(_K, _C - c0)) for c0 in range(0, _C, _K)]

_DX = np.float32(_IMG_W / _C)
_DY = np.float32(_IMG_H / _C)


def _grid_vecs():
    cell_x = np.tile(np.arange(_W, dtype=np.float32), _H)
    cell_y = np.repeat(np.arange(_H, dtype=np.float32), _W)
    return np.concatenate([_DX * cell_x, _DY * cell_y])


def _emit_row(c, j, v16, ib, ob, m, gxv, gyv):
    pos, anchor = c % 25, c // 25
    sl = pl.ds(v16, 16)
    p = ib[j, sl]
    if pos == 0 or pos >= 5:
        ob[j, sl] = p * m
    elif pos == 1:
        t = (_DX * p).astype(jnp.int32).astype(jnp.float32)
        ob[j, sl] = (t + gxv) * m
    elif pos == 2:
        t = (_DY * p).astype(jnp.int32).astype(jnp.float32)
        ob[j, sl] = (t + gyv) * m
    elif pos == 3:
        pw = float(_PRIOR_BOXES[anchor, 0])
        t = ((pw * p) * _IMG_W).astype(jnp.int32).astype(jnp.float32)
        ob[j, sl] = t * m
    else:
        ph = float(_PRIOR_BOXES[anchor, 1])
        t = ((ph * p) * _IMG_H).astype(jnp.int32).astype(jnp.float32)
        ob[j, sl] = t * m


def _sc_body(pred_hbm, y0_hbm, gxy_hbm, out_hbm,
             gx_v, gy_v, mask_v, in0, in1, out0, out1,
             sin0, sin1, sout0, sout1):
    wid = lax.axis_index("s") * 2 + lax.axis_index("c")
    pltpu.sync_copy(gxy_hbm.at[pl.ds(0, _HW)], gx_v)
    pltpu.sync_copy(gxy_hbm.at[pl.ds(_HW, _HW)], gy_v)
    in_bufs, in_sems = (in0, in1), (sin0, sin1)
    out_bufs, out_sems = (out0, out1), (sout0, sout1)
    nch = len(_CHUNKS)

    for bi in range(_BPW):
        b = wid * _BPW + bi
        ybase = pl.multiple_of(b * _HW, 8)

        pltpu.sync_copy(y0_hbm.at[pl.ds(ybase, _HW)], mask_v)

        def mask_body(v, _):
            sl = pl.ds(v * 16, 16)
            y = mask_v[sl]
            mask_v[sl] = _LAMBDA_OBJ * y + _LAMBDA_NONOBJ * jnp.negative(y + (-1.0))
            return 0

        lax.fori_loop(0, _NV, mask_body, 0)

        def in_cp(ch, buf, sem):
            c0, sz = _CHUNKS[ch]
            return pltpu.make_async_copy(
                pred_hbm.at[b, pl.ds(c0, sz)], buf.at[pl.ds(0, sz)], sem)

        def out_cp(ch, buf, sem):
            c0, sz = _CHUNKS[ch]
            return pltpu.make_async_copy(
                buf.at[pl.ds(0, sz)], out_hbm.at[b, pl.ds(c0, sz)], sem)

        in_cp(0, in_bufs[0], in_sems[0]).start()
        for ch in range(nch):
            cur = ch % 2
            c0, sz = _CHUNKS[ch]
            in_cp(ch, in_bufs[cur], in_sems[cur]).wait()
            if ch + 1 < nch:
                in_cp(ch + 1, in_bufs[1 - cur], in_sems[1 - cur]).start()
            if ch >= 2:
                out_cp(ch - 2, out_bufs[cur], out_sems[cur]).wait()
            ib, ob = in_bufs[cur], out_bufs[cur]
            rows = [c0 + j for j in range(sz)]
            need_g = any(c % 25 in (1, 2) for c in rows)

            def body(v, _, ib=ib, ob=ob, rows=rows, need_g=need_g):
                v16 = v * 16
                sl = pl.ds(v16, 16)
                m = mask_v[sl]
                gxv = gx_v[sl] if need_g else None
                gyv = gy_v[sl] if need_g else None
                for j, c in enumerate(rows):
                    _emit_row(c, j, v16, ib, ob, m, gxv, gyv)
                return 0

            lax.fori_loop(0, _NV, body, 0)
            out_cp(ch, out_bufs[cur], out_sems[cur]).start()
        out_cp(nch - 2, out_bufs[(nch - 2) % 2], out_sems[(nch - 2) % 2]).wait()
        out_cp(nch - 1, out_bufs[(nch - 1) % 2], out_sems[(nch - 1) % 2]).wait()


def kernel(pred, y_hat):
    B, C, H, W = pred.shape
    HW = H * W
    gxy = _grid_vecs()

    pred3 = pred.reshape(B, C, HW)
    y0 = y_hat[:, :, :, 0].reshape(B * HW)

    mesh = plsc.VectorSubcoreMesh(core_axis_name="c", subcore_axis_name="s")
    sc = pl.kernel(
        _sc_body,
        mesh=mesh,
        out_type=jax.ShapeDtypeStruct((B, C, HW), jnp.float32),
        scratch_types=[
            pltpu.VMEM((_HW,), jnp.float32),
            pltpu.VMEM((_HW,), jnp.float32),
            pltpu.VMEM((_HW,), jnp.float32),
            pltpu.VMEM((_K, _HW), jnp.float32),
            pltpu.VMEM((_K, _HW), jnp.float32),
            pltpu.VMEM((_K, _HW), jnp.float32),
            pltpu.VMEM((_K, _HW), jnp.float32),
            pltpu.SemaphoreType.DMA,
            pltpu.SemaphoreType.DMA,
            pltpu.SemaphoreType.DMA,
            pltpu.SemaphoreType.DMA,
        ],
    )
    out = sc(pred3, y0, jnp.asarray(gxy))
    return out.reshape(B, C, H, W)
